# baseline (speedup 1.0000x reference)
import functools

import jax
import jax.numpy as jnp
from jax import lax
from jax.experimental import pallas as pl
from jax.experimental.pallas import tpu as pltpu
from jax.experimental.pallas import tpu_sc as plsc

VOCAB = 1000000
EMB_D = 32
NC = 2
NS = 16
NW = NC * NS

N_TILE = VOCAB // 128
TAIL = VOCAB - N_TILE * 128
T128_ROWS = (VOCAB + 127) // 128 * 32
PER_W1 = N_TILE // NW
EXTRA1 = N_TILE - PER_W1 * NW

_CP = pltpu.CompilerParams(
    needs_layout_passes=False, disable_bounds_checks=True
)


def _worker_id():
    return lax.axis_index("s") * NC + lax.axis_index("c")


@functools.cache
def _build_transpose():
    mesh = plsc.VectorSubcoreMesh(core_axis_name="c", subcore_axis_name="s")

    @functools.partial(
        pl.kernel,
        mesh=mesh,
        compiler_params=_CP,
        out_type=jax.ShapeDtypeStruct((T128_ROWS, 128), jnp.float32),
        scratch_types=[
            pltpu.VMEM((32, 129), jnp.float32),
            pltpu.VMEM((32, 129), jnp.float32),
            pltpu.VMEM((32, 128), jnp.float32),
            pltpu.VMEM((32, 128), jnp.float32),
            pltpu.VMEM((32, 64), jnp.float32),
            pltpu.VMEM((16, 128), jnp.float32),
            pltpu.SemaphoreType.DMA,
            pltpu.SemaphoreType.DMA,
            pltpu.SemaphoreType.DMA,
            pltpu.SemaphoreType.DMA,
        ],
    )
    def k1(src_hbm, out_hbm, buf0, buf1, dst0, dst1, tbuf, tdst,
           si0, si1, so0, so1):
        w = _worker_id()
        start = w * PER_W1
        iota = lax.iota(jnp.int32, 16)
        dlo = iota * 129
        dhi = dlo + 16 * 129

        def in_copies(c, buf, sem):
            c_off = pl.multiple_of(c * 128, 128)
            return [
                pltpu.make_async_copy(
                    src_hbm.at[pl.ds(dblk * 8, 8), pl.ds(c_off, 128)],
                    buf.at[pl.ds(dblk * 8, 8), pl.ds(0, 128)],
                    sem,
                )
                for dblk in range(4)
            ]

        def out_copy(c, dst, sem):
            r_off = pl.multiple_of(c * 32, 32)
            return pltpu.make_async_copy(
                dst, out_hbm.at[pl.ds(r_off, 32), :], sem
            )

        def shuffle(buf, dst, n_r=32):
            ihi = iota + 16

            def rbody(r, _):
                for q in range(8):
                    dvec = ihi if q % 2 else iota
                    isp = jnp.full((16,), 4 * r + q // 2, jnp.int32)
                    dst[r, pl.ds(16 * q, 16)] = plsc.load_gather(buf, [dvec, isp])
                return 0

            lax.fori_loop(0, n_r, rbody, 0, unroll=False)

        for cp in in_copies(start, buf0, si0):
            cp.start()

        def body(u, _):
            t0 = start + 2 * u
            t1 = t0 + 1

            for cp in in_copies(t1, buf1, si1):
                cp.start()
            for cp in in_copies(t0, buf0, si0):
                cp.wait()

            @pl.when(u >= 1)
            def _():
                out_copy(t0 - 2, dst0, so0).wait()

            shuffle(buf0, dst0)
            out_copy(t0, dst0, so0).start()

            @pl.when(u + 1 < PER_W1 // 2)
            def _():
                for cp in in_copies(t0 + 2, buf0, si0):
                    cp.start()

            for cp in in_copies(t1, buf1, si1):
                cp.wait()

            @pl.when(u >= 1)
            def _():
                out_copy(t1 - 2, dst1, so1).wait()

            shuffle(buf1, dst1)
            out_copy(t1, dst1, so1).start()
            return 0

        lax.fori_loop(0, PER_W1 // 2, body, 0, unroll=False)

        out_copy(start + PER_W1 - 2, dst0, so0).wait()
        out_copy(start + PER_W1 - 1, dst1, so1).wait()

        @pl.when(w < EXTRA1)
        def _():
            c = NW * PER_W1 + w
            for cp in in_copies(c, buf0, si0):
                cp.start()
            for cp in in_copies(c, buf0, si0):
                cp.wait()
            shuffle(buf0, dst0)
            out_copy(c, dst0, so0).start()
            out_copy(c, dst0, so0).wait()

        @pl.when(w == EXTRA1)
        def _():
            for dblk in range(4):
                pltpu.sync_copy(
                    src_hbm.at[pl.ds(dblk * 8, 8), pl.ds(N_TILE * 128, TAIL)],
                    tbuf.at[pl.ds(dblk * 8, 8), :],
                )
            shuffle(tbuf, tdst, n_r=16)
            pltpu.sync_copy(tdst, out_hbm.at[pl.ds(N_TILE * 32, 16), :])

    return k1


@functools.cache
def _build_gather(n_batch: int, n_fields: int):
    mesh = plsc.VectorSubcoreMesh(core_axis_name="c", subcore_axis_name="s")
    n_bblk = n_batch // 128
    n_blocks = n_fields * n_bblk
    assert n_blocks % (2 * NW) == 0
    per_w = n_blocks // NW
    n_pair = per_w // 2

    @functools.partial(
        pl.kernel,
        mesh=mesh,
        compiler_params=_CP,
        out_type=jax.ShapeDtypeStruct((n_fields * 4 * n_bblk, 8, 128), jnp.float32),
        scratch_types=[
            pltpu.VMEM((256,), jnp.int32),
            pltpu.VMEM((256,), jnp.int32),
            pltpu.VMEM((128,), jnp.int32),
            pltpu.VMEM((128,), jnp.int32),
            pltpu.VMEM((128,), jnp.int32),
            pltpu.VMEM((128,), jnp.int32),
            pltpu.VMEM((128, 129), jnp.float32),
            pltpu.VMEM((128, 129), jnp.float32),
            pltpu.VMEM((32, 128), jnp.float32),
            pltpu.VMEM((32, 128), jnp.float32),
            pltpu.SemaphoreType.DMA,
            pltpu.SemaphoreType.DMA,
            pltpu.SemaphoreType.DMA,
            pltpu.SemaphoreType.DMA,
            pltpu.SemaphoreType.DMA,
        ],
    )
    def k2(t128_hbm, idx_hbm, out_hbm, idxa, idxb, rowv0, rowv1,
           colv0, colv1, rows0, rows1, dst0, dst1,
           sem_i, sg0, sg1, so0, so1):
        w = _worker_id()
        start = w * per_w
        iota = lax.iota(jnp.int32, 16)

        def idx_copy(u):
            blk = start + 2 * u
            f = blk // n_bblk
            b_off = pl.multiple_of((blk % n_bblk) * 128, 128)
            return pltpu.make_async_copy(
                idx_hbm.at[f, pl.ds(b_off, 256)], idxb, sem_i
            )

        def gather_copy(rowv, rows, sem):
            return pltpu.make_async_copy(
                t128_hbm.at[rowv], rows.at[:, pl.ds(0, 128)], sem
            )

        def out_copies(t, dst, sem):
            blk = start + t
            f = blk // n_bblk
            bb = blk % n_bblk
            return [
                pltpu.make_async_copy(
                    dst.at[pl.ds(dblk * 8, 8), :],
                    out_hbm.at[(f * 4 + dblk) * n_bblk + bb],
                    sem,
                )
                for dblk in range(4)
            ]

        def prep(half, rowv, colv):
            for j in range(8):
                iv = idxa[pl.ds(half * 128 + j * 16, 16)]
                rowv[pl.ds(j * 16, 16)] = lax.shift_right_logical(iv, 2)
                colv[pl.ds(j * 16, 16)] = lax.shift_left(iv & 3, 5)

        def select(rows, colv, dst):
            for j in range(8):
                bvec = iota + j * 16
                cb = colv[pl.ds(j * 16, 16)]
                for d in range(32):
                    v = plsc.load_gather(rows, [bvec, cb + d])
                    dst[d, pl.ds(j * 16, 16)] = v

        idx_copy(0).start()
        idx_copy(0).wait()
        for j in range(16):
            idxa[pl.ds(j * 16, 16)] = idxb[pl.ds(j * 16, 16)]
        prep(0, rowv0, colv0)
        gather_copy(rowv0, rows0, sg0).start()

        def body(u, _):
            t0 = 2 * u
            t1 = t0 + 1

            @pl.when(u + 1 < n_pair)
            def _():
                idx_copy(u + 1).start()

            prep(1, rowv1, colv1)
            gather_copy(rowv1, rows1, sg1).start()

            gather_copy(rowv0, rows0, sg0).wait()

            @pl.when(u >= 1)
            def _():
                for cp in out_copies(t0 - 2, dst0, so0):
                    cp.wait()

            select(rows0, colv0, dst0)
            for cp in out_copies(t0, dst0, so0):
                cp.start()

            @pl.when(u + 1 < n_pair)
            def _():
                idx_copy(u + 1).wait()
                for j in range(16):
                    idxa[pl.ds(j * 16, 16)] = idxb[pl.ds(j * 16, 16)]
                prep(0, rowv0, colv0)
                gather_copy(rowv0, rows0, sg0).start()

            gather_copy(rowv1, rows1, sg1).wait()

            @pl.when(u >= 1)
            def _():
                for cp in out_copies(t1 - 2, dst1, so1):
                    cp.wait()

            select(rows1, colv1, dst1)
            for cp in out_copies(t1, dst1, so1):
                cp.start()
            return 0

        lax.fori_loop(0, n_pair, body, 0, unroll=False)

        for cp in out_copies(per_w - 2, dst0, so0):
            cp.wait()
        for cp in out_copies(per_w - 1, dst1, so1):
            cp.wait()

    return k2


def kernel(input, embedding):
    b, f = input.shape
    idx_t = input.T.astype(jnp.int32)
    table_t = embedding.T
    t128 = _build_transpose()(table_t)
    out_t = _build_gather(b, f)(t128, idx_t)
    out5 = out_t.reshape(f, 4, b // 128, 8, 128)
    out = out5.transpose(2, 4, 0, 1, 3).reshape(b, f, EMB_D)
    return out


# speedup vs baseline: 2.3240x; 2.3240x over previous
import functools

import jax
import jax.numpy as jnp
from jax import lax
from jax.experimental import pallas as pl
from jax.experimental.pallas import tpu as pltpu
from jax.experimental.pallas import tpu_sc as plsc

VOCAB = 1000000
EMB_D = 32
NC = 2
NS = 16
NW = NC * NS

N_TILE = VOCAB // 128
TAIL = VOCAB - N_TILE * 128
T128_ROWS = (VOCAB + 127) // 128 * 32
PER_W1 = N_TILE // NW
EXTRA1 = N_TILE - PER_W1 * NW

_CP = pltpu.CompilerParams(
    needs_layout_passes=False, disable_bounds_checks=True
)


def _worker_id():
    return lax.axis_index("s") * NC + lax.axis_index("c")


@functools.cache
def _build_transpose():
    mesh = plsc.VectorSubcoreMesh(core_axis_name="c", subcore_axis_name="s")

    @functools.partial(
        pl.kernel,
        mesh=mesh,
        compiler_params=_CP,
        out_type=jax.ShapeDtypeStruct((T128_ROWS, 128), jnp.float32),
        scratch_types=[
            pltpu.VMEM((32, 129), jnp.float32),
            pltpu.VMEM((32, 129), jnp.float32),
            pltpu.VMEM((32, 128), jnp.float32),
            pltpu.VMEM((32, 128), jnp.float32),
            pltpu.VMEM((32, 64), jnp.float32),
            pltpu.VMEM((16, 128), jnp.float32),
            pltpu.VMEM((16,), jnp.int32),
            pltpu.SemaphoreType.DMA,
            pltpu.SemaphoreType.DMA,
            pltpu.SemaphoreType.DMA,
            pltpu.SemaphoreType.DMA,
        ],
    )
    def k1(src_hbm, out_hbm, buf0, buf1, dst0, dst1, tbuf, tdst, zscr,
           si0, si1, so0, so1):
        w = _worker_id()
        start = w * PER_W1
        iota = lax.iota(jnp.int32, 16)
        dlo = iota * 129
        dhi = dlo + 16 * 129

        def in_copies(c, buf, sem):
            c_off = pl.multiple_of(c * 128, 128)
            return [
                pltpu.make_async_copy(
                    src_hbm.at[pl.ds(dblk * 8, 8), pl.ds(c_off, 128)],
                    buf.at[pl.ds(dblk * 8, 8), pl.ds(0, 128)],
                    sem,
                )
                for dblk in range(4)
            ]

        def out_copy(c, dst, sem):
            r_off = pl.multiple_of(c * 32, 32)
            return pltpu.make_async_copy(
                dst, out_hbm.at[pl.ds(r_off, 32), :], sem
            )

        ihi = iota + 16

        def shuffle(buf, dst, n_r=32):
            @plsc.parallel_loop(0, n_r, 1, unroll=4)
            def _(r):
                base = r * 4
                for q in range(8):
                    dvec = ihi if q % 2 else iota
                    isp = jnp.full((16,), base + q // 2, jnp.int32)
                    dst[r, pl.ds(16 * q, 16)] = plsc.load_gather(buf, [dvec, isp])

        for cp in in_copies(start, buf0, si0):
            cp.start()

        def body(u, _):
            t0 = start + 2 * u
            t1 = t0 + 1

            for cp in in_copies(t1, buf1, si1):
                cp.start()
            for cp in in_copies(t0, buf0, si0):
                cp.wait()

            @pl.when(u >= 1)
            def _():
                out_copy(t0 - 2, dst0, so0).wait()

            shuffle(buf0, dst0)
            out_copy(t0, dst0, so0).start()

            @pl.when(u + 1 < PER_W1 // 2)
            def _():
                for cp in in_copies(t0 + 2, buf0, si0):
                    cp.start()

            for cp in in_copies(t1, buf1, si1):
                cp.wait()

            @pl.when(u >= 1)
            def _():
                out_copy(t1 - 2, dst1, so1).wait()

            shuffle(buf1, dst1)
            out_copy(t1, dst1, so1).start()
            return 0

        lax.fori_loop(0, PER_W1 // 2, body, 0, unroll=False)

        out_copy(start + PER_W1 - 2, dst0, so0).wait()
        out_copy(start + PER_W1 - 1, dst1, so1).wait()

        @pl.when(w < EXTRA1)
        def _():
            c = NW * PER_W1 + w
            for cp in in_copies(c, buf0, si0):
                cp.start()
            for cp in in_copies(c, buf0, si0):
                cp.wait()
            shuffle(buf0, dst0)
            out_copy(c, dst0, so0).start()
            out_copy(c, dst0, so0).wait()

        @pl.when(w == EXTRA1)
        def _():
            for dblk in range(4):
                pltpu.sync_copy(
                    src_hbm.at[pl.ds(dblk * 8, 8), pl.ds(N_TILE * 128, TAIL)],
                    tbuf.at[pl.ds(dblk * 8, 8), :],
                )
            shuffle(tbuf, tdst, n_r=16)
            pltpu.sync_copy(tdst, out_hbm.at[pl.ds(N_TILE * 32, 16), :])

    return k1


@functools.cache
def _build_gather(n_batch: int, n_fields: int):
    mesh = plsc.VectorSubcoreMesh(core_axis_name="c", subcore_axis_name="s")
    n_bblk = n_batch // 128
    n_blocks = n_fields * n_bblk
    assert n_blocks % (2 * NW) == 0
    per_w = n_blocks // NW
    n_pair = per_w // 2

    @functools.partial(
        pl.kernel,
        mesh=mesh,
        compiler_params=pltpu.CompilerParams(
            needs_layout_passes=False,
            disable_bounds_checks=True,
            use_tc_tiling_on_sc=False,
        ),
        out_type=jax.ShapeDtypeStruct((n_fields * 4 * n_bblk, 8, 128), jnp.float32),
        scratch_types=[
            pltpu.VMEM((256,), jnp.int32),
            pltpu.VMEM((256,), jnp.int32),
            pltpu.VMEM((128,), jnp.int32),
            pltpu.VMEM((128,), jnp.int32),
            pltpu.VMEM((128, 32), jnp.float32),
            pltpu.VMEM((128, 32), jnp.float32),
            pltpu.VMEM((32, 129), jnp.float32),
            pltpu.VMEM((32, 129), jnp.float32),
            pltpu.SemaphoreType.DMA,
            pltpu.SemaphoreType.DMA,
            pltpu.SemaphoreType.DMA,
            pltpu.SemaphoreType.DMA,
            pltpu.SemaphoreType.DMA,
        ],
    )
    def k2(t128_hbm, idx_hbm, out_hbm, idxa, idxb, rowv0, rowv1,
           rows0, rows1, dst0, dst1, sem_i, sg0, sg1, so0, so1):
        w = _worker_id()
        start = w * per_w
        iota = lax.iota(jnp.int32, 16)
        ihi = iota + 16

        def idx_copy(u):
            blk = start + 2 * u
            f = blk // n_bblk
            b_off = pl.multiple_of((blk % n_bblk) * 128, 128)
            return pltpu.make_async_copy(
                idx_hbm.at[f, pl.ds(b_off, 256)], idxb, sem_i
            )

        def gather_copy(rowv, rows, sem):
            return pltpu.make_async_copy(t128_hbm.at[rowv], rows, sem)

        def out_copies(t, dst, sem):
            blk = start + t
            f = blk // n_bblk
            bb = blk % n_bblk
            return [
                pltpu.make_async_copy(
                    dst.at[pl.ds(dblk * 8, 8), pl.ds(0, 128)],
                    out_hbm.at[(f * 4 + dblk) * n_bblk + bb],
                    sem,
                )
                for dblk in range(4)
            ]

        def prep(half, rowv):
            for j in range(8):
                rowv[pl.ds(j * 16, 16)] = idxa[pl.ds(half * 128 + j * 16, 16)]

        idx_copy(0).start()
        idx_copy(0).wait()
        for j in range(16):
            idxa[pl.ds(j * 16, 16)] = idxb[pl.ds(j * 16, 16)]
        prep(0, rowv0)
        gather_copy(rowv0, rows0, sg0).start()

        def select(rows, dst):
            @plsc.parallel_loop(0, 128, 1, unroll=4)
            def _(b):
                bsp = jnp.full((16,), b, jnp.int32)
                plsc.store_scatter(dst, [iota, bsp], rows[b, pl.ds(0, 16)])
                plsc.store_scatter(dst, [ihi, bsp], rows[b, pl.ds(16, 16)])

        def body(u, _):
            t0 = 2 * u
            t1 = t0 + 1

            @pl.when(u + 1 < n_pair)
            def _():
                idx_copy(u + 1).start()

            prep(1, rowv1)
            gather_copy(rowv1, rows1, sg1).start()

            gather_copy(rowv0, rows0, sg0).wait()

            @pl.when(u >= 1)
            def _():
                for cp in out_copies(t0 - 2, dst0, so0):
                    cp.wait()

            select(rows0, dst0)
            for cp in out_copies(t0, dst0, so0):
                cp.start()

            @pl.when(u + 1 < n_pair)
            def _():
                idx_copy(u + 1).wait()
                for j in range(16):
                    idxa[pl.ds(j * 16, 16)] = idxb[pl.ds(j * 16, 16)]
                prep(0, rowv0)
                gather_copy(rowv0, rows0, sg0).start()

            gather_copy(rowv1, rows1, sg1).wait()

            @pl.when(u >= 1)
            def _():
                for cp in out_copies(t1 - 2, dst1, so1):
                    cp.wait()

            select(rows1, dst1)
            for cp in out_copies(t1, dst1, so1):
                cp.start()
            return 0

        lax.fori_loop(0, n_pair, body, 0, unroll=False)

        for cp in out_copies(per_w - 2, dst0, so0):
            cp.wait()
        for cp in out_copies(per_w - 1, dst1, so1):
            cp.wait()

    return k2


def kernel(input, embedding):
    b, f = input.shape
    idx_t = input.T.astype(jnp.int32)
    table_t = embedding.T
    t128 = _build_transpose()(table_t)
    t_rows = t128.reshape(T128_ROWS * 4, EMB_D)
    out_t = _build_gather(b, f)(t_rows, idx_t)
    out5 = out_t.reshape(f, 4, b // 128, 8, 128)
    out = out5.transpose(2, 4, 0, 1, 3).reshape(b, f, EMB_D)
    return out


# speedup vs baseline: 2.3265x; 1.0011x over previous
import functools

import jax
import jax.numpy as jnp
from jax import lax
from jax.experimental import pallas as pl
from jax.experimental.pallas import tpu as pltpu
from jax.experimental.pallas import tpu_sc as plsc

VOCAB = 1000000
EMB_D = 32
NC = 2
NS = 16
NW = NC * NS

N_TILE = VOCAB // 128
TAIL = VOCAB - N_TILE * 128
T128_ROWS = (VOCAB + 127) // 128 * 32
PER_W1 = N_TILE // NW
EXTRA1 = N_TILE - PER_W1 * NW

_CP = pltpu.CompilerParams(
    needs_layout_passes=False, disable_bounds_checks=True
)


def _worker_id():
    return lax.axis_index("s") * NC + lax.axis_index("c")


@functools.cache
def _build_transpose():
    mesh = plsc.VectorSubcoreMesh(core_axis_name="c", subcore_axis_name="s")

    @functools.partial(
        pl.kernel,
        mesh=mesh,
        compiler_params=_CP,
        out_type=jax.ShapeDtypeStruct((T128_ROWS, 128), jnp.float32),
        scratch_types=[
            pltpu.VMEM((32, 129), jnp.float32),
            pltpu.VMEM((32, 129), jnp.float32),
            pltpu.VMEM((32, 128), jnp.float32),
            pltpu.VMEM((32, 128), jnp.float32),
            pltpu.VMEM((32, 64), jnp.float32),
            pltpu.VMEM((16, 128), jnp.float32),
            pltpu.VMEM((16,), jnp.int32),
            pltpu.SemaphoreType.DMA,
            pltpu.SemaphoreType.DMA,
            pltpu.SemaphoreType.DMA,
            pltpu.SemaphoreType.DMA,
        ],
    )
    def k1(src_hbm, out_hbm, buf0, buf1, dst0, dst1, tbuf, tdst, zscr,
           si0, si1, so0, so1):
        w = _worker_id()
        start = w * PER_W1
        iota = lax.iota(jnp.int32, 16)
        dlo = iota * 129
        dhi = dlo + 16 * 129

        def in_copies(c, buf, sem):
            c_off = pl.multiple_of(c * 128, 128)
            return [
                pltpu.make_async_copy(
                    src_hbm.at[pl.ds(dblk * 8, 8), pl.ds(c_off, 128)],
                    buf.at[pl.ds(dblk * 8, 8), pl.ds(0, 128)],
                    sem,
                )
                for dblk in range(4)
            ]

        def out_copy(c, dst, sem):
            r_off = pl.multiple_of(c * 32, 32)
            return pltpu.make_async_copy(
                dst, out_hbm.at[pl.ds(r_off, 32), :], sem
            )

        ihi = iota + 16

        def shuffle(buf, dst, n_r=32):
            @plsc.parallel_loop(0, n_r, 1, unroll=8)
            def _(r):
                base = r * 4
                for q in range(8):
                    dvec = ihi if q % 2 else iota
                    isp = jnp.full((16,), base + q // 2, jnp.int32)
                    dst[r, pl.ds(16 * q, 16)] = plsc.load_gather(buf, [dvec, isp])

        for cp in in_copies(start, buf0, si0):
            cp.start()

        def body(u, _):
            t0 = start + 2 * u
            t1 = t0 + 1

            for cp in in_copies(t1, buf1, si1):
                cp.start()
            for cp in in_copies(t0, buf0, si0):
                cp.wait()

            @pl.when(u >= 1)
            def _():
                out_copy(t0 - 2, dst0, so0).wait()

            shuffle(buf0, dst0)
            out_copy(t0, dst0, so0).start()

            @pl.when(u + 1 < PER_W1 // 2)
            def _():
                for cp in in_copies(t0 + 2, buf0, si0):
                    cp.start()

            for cp in in_copies(t1, buf1, si1):
                cp.wait()

            @pl.when(u >= 1)
            def _():
                out_copy(t1 - 2, dst1, so1).wait()

            shuffle(buf1, dst1)
            out_copy(t1, dst1, so1).start()
            return 0

        lax.fori_loop(0, PER_W1 // 2, body, 0, unroll=False)

        out_copy(start + PER_W1 - 2, dst0, so0).wait()
        out_copy(start + PER_W1 - 1, dst1, so1).wait()

        @pl.when(w < EXTRA1)
        def _():
            c = NW * PER_W1 + w
            for cp in in_copies(c, buf0, si0):
                cp.start()
            for cp in in_copies(c, buf0, si0):
                cp.wait()
            shuffle(buf0, dst0)
            out_copy(c, dst0, so0).start()
            out_copy(c, dst0, so0).wait()

        @pl.when(w == EXTRA1)
        def _():
            for dblk in range(4):
                pltpu.sync_copy(
                    src_hbm.at[pl.ds(dblk * 8, 8), pl.ds(N_TILE * 128, TAIL)],
                    tbuf.at[pl.ds(dblk * 8, 8), :],
                )
            shuffle(tbuf, tdst, n_r=16)
            pltpu.sync_copy(tdst, out_hbm.at[pl.ds(N_TILE * 32, 16), :])

    return k1


@functools.cache
def _build_gather(n_batch: int, n_fields: int):
    mesh = plsc.VectorSubcoreMesh(core_axis_name="c", subcore_axis_name="s")
    n_bblk = n_batch // 128
    n_blocks = n_fields * n_bblk
    assert n_blocks % (2 * NW) == 0
    per_w = n_blocks // NW
    n_pair = per_w // 2

    @functools.partial(
        pl.kernel,
        mesh=mesh,
        compiler_params=pltpu.CompilerParams(
            needs_layout_passes=False,
            disable_bounds_checks=True,
            use_tc_tiling_on_sc=False,
        ),
        out_type=jax.ShapeDtypeStruct((n_fields * 4 * n_bblk, 8, 128), jnp.float32),
        scratch_types=[
            pltpu.VMEM((256,), jnp.int32),
            pltpu.VMEM((256,), jnp.int32),
            pltpu.VMEM((128,), jnp.int32),
            pltpu.VMEM((128,), jnp.int32),
            pltpu.VMEM((128, 32), jnp.float32),
            pltpu.VMEM((128, 32), jnp.float32),
            pltpu.VMEM((32, 129), jnp.float32),
            pltpu.VMEM((32, 129), jnp.float32),
            pltpu.SemaphoreType.DMA,
            pltpu.SemaphoreType.DMA,
            pltpu.SemaphoreType.DMA,
            pltpu.SemaphoreType.DMA,
            pltpu.SemaphoreType.DMA,
        ],
    )
    def k2(t128_hbm, idx_hbm, out_hbm, idxa, idxb, rowv0, rowv1,
           rows0, rows1, dst0, dst1, sem_i, sg0, sg1, so0, so1):
        w = _worker_id()
        start = w * per_w
        iota = lax.iota(jnp.int32, 16)
        ihi = iota + 16

        def idx_copy(u):
            blk = start + 2 * u
            f = blk // n_bblk
            b_off = pl.multiple_of((blk % n_bblk) * 128, 128)
            return pltpu.make_async_copy(
                idx_hbm.at[f, pl.ds(b_off, 256)], idxb, sem_i
            )

        def gather_copy(rowv, rows, sem):
            return pltpu.make_async_copy(t128_hbm.at[rowv], rows, sem)

        def out_copies(t, dst, sem):
            blk = start + t
            f = blk // n_bblk
            bb = blk % n_bblk
            return [
                pltpu.make_async_copy(
                    dst.at[pl.ds(dblk * 8, 8), pl.ds(0, 128)],
                    out_hbm.at[(f * 4 + dblk) * n_bblk + bb],
                    sem,
                )
                for dblk in range(4)
            ]

        def prep(half, rowv):
            for j in range(8):
                rowv[pl.ds(j * 16, 16)] = idxa[pl.ds(half * 128 + j * 16, 16)]

        idx_copy(0).start()
        idx_copy(0).wait()
        for j in range(16):
            idxa[pl.ds(j * 16, 16)] = idxb[pl.ds(j * 16, 16)]
        prep(0, rowv0)
        gather_copy(rowv0, rows0, sg0).start()

        def select(rows, dst):
            @plsc.parallel_loop(0, 128, 1, unroll=4)
            def _(b):
                bsp = jnp.full((16,), b, jnp.int32)
                plsc.store_scatter(dst, [iota, bsp], rows[b, pl.ds(0, 16)])
                plsc.store_scatter(dst, [ihi, bsp], rows[b, pl.ds(16, 16)])

        def body(u, _):
            t0 = 2 * u
            t1 = t0 + 1

            @pl.when(u + 1 < n_pair)
            def _():
                idx_copy(u + 1).start()

            prep(1, rowv1)
            gather_copy(rowv1, rows1, sg1).start()

            gather_copy(rowv0, rows0, sg0).wait()

            @pl.when(u >= 1)
            def _():
                for cp in out_copies(t0 - 2, dst0, so0):
                    cp.wait()

            select(rows0, dst0)
            for cp in out_copies(t0, dst0, so0):
                cp.start()

            @pl.when(u + 1 < n_pair)
            def _():
                idx_copy(u + 1).wait()
                for j in range(16):
                    idxa[pl.ds(j * 16, 16)] = idxb[pl.ds(j * 16, 16)]
                prep(0, rowv0)
                gather_copy(rowv0, rows0, sg0).start()

            gather_copy(rowv1, rows1, sg1).wait()

            @pl.when(u >= 1)
            def _():
                for cp in out_copies(t1 - 2, dst1, so1):
                    cp.wait()

            select(rows1, dst1)
            for cp in out_copies(t1, dst1, so1):
                cp.start()
            return 0

        lax.fori_loop(0, n_pair, body, 0, unroll=False)

        for cp in out_copies(per_w - 2, dst0, so0):
            cp.wait()
        for cp in out_copies(per_w - 1, dst1, so1):
            cp.wait()

    return k2


def kernel(input, embedding):
    b, f = input.shape
    idx_t = input.T.astype(jnp.int32)
    table_t = embedding.T
    t128 = _build_transpose()(table_t)
    t_rows = t128.reshape(T128_ROWS * 4, EMB_D)
    out_t = _build_gather(b, f)(t_rows, idx_t)
    out5 = out_t.reshape(f, 4, b // 128, 8, 128)
    out = out5.transpose(2, 4, 0, 1, 3).reshape(b, f, EMB_D)
    return out
